# Initial kernel scaffold; baseline (speedup 1.0000x reference)
import jax
import jax.numpy as jnp
from jax.experimental import pallas as pl


def kernel(e_t, H, r_q, r_space, e_space, action_mask, W1_w, W1_b, W2_w, W2_b, rel_table, ent_table):
    raise NotImplementedError("write your pallas kernel here")


# speedup vs baseline: 3.4079x; 3.4079x over previous
import functools

import jax
import jax.numpy as jnp
import numpy as np
from jax import lax
from jax.experimental import pallas as pl
from jax.experimental.pallas import tpu as pltpu
from jax.experimental.pallas import tpu_sc as plsc

B, A = 4096, 200
ENT_DIM, REL_DIM, HIST_DIM = 64, 64, 128
N_ENT, N_REL = 1000000, 1000
ACTION_DIM = ENT_DIM + REL_DIM
HUGE = 1e9

BLK = 512


def _prologue_body(e_ref, h_ref, rq_ref, w1_ref, b1_ref, w2_ref, b2_ref,
                   rel_ref, x2e_ref, p_ref):
    X = jnp.concatenate([e_ref[...], h_ref[...], rq_ref[...]], axis=-1)
    X = lax.dot_general(X, w1_ref[...], (((1,), (1,)), ((), ()))) + b1_ref[...]
    X = jax.nn.relu(X)
    X2 = lax.dot_general(X, w2_ref[...], (((1,), (1,)), ((), ()))) + b2_ref[...]
    x2e_ref[...] = X2[:, ENT_DIM:]
    p_ref[...] = lax.dot_general(X2[:, :REL_DIM], rel_ref[...],
                                 (((1,), (1,)), ((), ())))


_prologue = pl.pallas_call(
    _prologue_body,
    grid=(B // BLK,),
    in_specs=[
        pl.BlockSpec((BLK, ENT_DIM), lambda i: (i, 0)),
        pl.BlockSpec((BLK, HIST_DIM), lambda i: (i, 0)),
        pl.BlockSpec((BLK, REL_DIM), lambda i: (i, 0)),
        pl.BlockSpec((ACTION_DIM, ACTION_DIM + HIST_DIM), lambda i: (0, 0)),
        pl.BlockSpec((1, ACTION_DIM), lambda i: (0, 0)),
        pl.BlockSpec((ACTION_DIM, ACTION_DIM), lambda i: (0, 0)),
        pl.BlockSpec((1, ACTION_DIM), lambda i: (0, 0)),
        pl.BlockSpec((N_REL, REL_DIM), lambda i: (0, 0)),
    ],
    out_specs=[
        pl.BlockSpec((BLK, ENT_DIM), lambda i: (i, 0)),
        pl.BlockSpec((BLK, N_REL), lambda i: (i, 0)),
    ],
    out_shape=[
        jax.ShapeDtypeStruct((B, ENT_DIM), jnp.float32),
        jax.ShapeDtypeStruct((B, N_REL), jnp.float32),
    ],
)

NC, NS, L = 2, 16, 16
NW = NC * NS
BPW = B // NW
CH = 16
NCH = BPW // CH
G1, G2 = 128, 72


def _round_bf16(v):
    b = plsc.bitcast(v, jnp.uint32)
    r = (b + jnp.uint32(0x7FFF) + ((b >> 16) & jnp.uint32(1))) \
        & jnp.uint32(0xFFFF0000)
    return plsc.bitcast(r, jnp.float32)


def _sc_body(x2e_hbm, p_hbm, es_hbm, rs_hbm, ent_hbm, scores_hbm,
             x2e_v, p_v, eidx_v, ridx_v, rows_v, sbuf_v, gsem):
    wid = lax.axis_index("s") * NC + lax.axis_index("c")
    b0 = wid * BPW
    lane = lax.iota(jnp.int32, L)

    def chunk_body(ch, _):
        bc = b0 + ch * CH
        pltpu.sync_copy(x2e_hbm.at[pl.ds(bc, CH)], x2e_v)
        pltpu.sync_copy(p_hbm.at[pl.ds(bc, CH)], p_v)
        pltpu.sync_copy(es_hbm.at[pl.ds(bc, CH)], eidx_v)
        pltpu.sync_copy(rs_hbm.at[pl.ds(bc, CH)], ridx_v)

        def b_body(bb, _):
            cp0 = pltpu.async_copy(
                ent_hbm.at[eidx_v.at[bb, pl.ds(0, G1)]],
                rows_v.at[pl.ds(0, G1)], gsem)
            cp1 = pltpu.async_copy(
                ent_hbm.at[eidx_v.at[bb, pl.ds(G1, G2)]],
                rows_v.at[pl.ds(G1, G2)], gsem)
            cp0.wait()
            cp1.wait()

            xe = [_round_bf16(x2e_v[bb, pl.ds(k * L, L)])
                  for k in range(ENT_DIM // L)]

            def a_body(j, _):
                aoff = jnp.minimum(j * L, A - L)
                s_acc = jnp.zeros((L,), jnp.float32)
                for t in range(L):
                    part = _round_bf16(rows_v[aoff + t, pl.ds(0, L)]) * xe[0]
                    for k in range(1, ENT_DIM // L):
                        part = part + _round_bf16(
                            rows_v[aoff + t, pl.ds(k * L, L)]) * xe[k]
                    tot = jnp.sum(part)
                    s_acc = jnp.where(lane == t, tot, s_acc)
                ridx16 = ridx_v[bb, pl.ds(aoff, L)]
                bvec = jnp.full((L,), bb, jnp.int32)
                prel = plsc.load_gather(p_v, [bvec, ridx16])
                sbuf_v[pl.ds(aoff, L)] = s_acc + prel
                return 0

            lax.fori_loop(0, (A + L - 1) // L, a_body, 0)
            pltpu.sync_copy(sbuf_v, scores_hbm.at[bc + bb])
            return 0

        lax.fori_loop(0, CH, b_body, 0)
        return 0

    lax.fori_loop(0, NCH, chunk_body, 0)


_sc_scores = pl.kernel(
    _sc_body,
    out_type=jax.ShapeDtypeStruct((B, A), jnp.float32),
    mesh=plsc.VectorSubcoreMesh(core_axis_name="c", subcore_axis_name="s"),
    compiler_params=pltpu.CompilerParams(needs_layout_passes=False,
                                         use_tc_tiling_on_sc=False),
    scratch_types=[
        pltpu.VMEM((CH, ENT_DIM), jnp.float32),
        pltpu.VMEM((CH, N_REL), jnp.float32),
        pltpu.VMEM((CH, A), jnp.int32),
        pltpu.VMEM((CH, A), jnp.int32),
        pltpu.VMEM((A, ENT_DIM), jnp.float32),
        pltpu.VMEM((A,), jnp.float32),
        pltpu.SemaphoreType.DMA,
    ],
)

EBLK = 512


def _threefry_bits(n0):
    k0 = jnp.uint32(0)
    k1 = jnp.uint32(42)
    ks2 = jnp.uint32(0x1BD11BDA) ^ k0 ^ k1
    rot = ((13, 15, 26, 6), (17, 29, 16, 24))
    x0 = jnp.zeros_like(n0) + k0
    x1 = n0 + k1
    ks = ((k1, ks2), (ks2, k0), (k0, k1), (k1, ks2), (ks2, k0))
    for i in range(5):
        for r in rot[i % 2]:
            x0 = x0 + x1
            x1 = (x1 << r) | (x1 >> (32 - r))
            x1 = x1 ^ x0
        x0 = x0 + ks[i][0]
        x1 = x1 + ks[i][1] + jnp.uint32(i + 1)
    return x0 ^ x1


def _epilogue_body(s_ref, rs_ref, es_ref, m_ref, ap_ref, nr_ref, ne_ref,
                   ent_ref):
    i = pl.program_id(0)
    scores = s_ref[...]
    mask = m_ref[...].astype(jnp.float32)
    masked = scores - (1.0 - mask) * HUGE

    rows = jax.lax.broadcasted_iota(jnp.uint32, (EBLK, A), 0)
    cols = jax.lax.broadcasted_iota(jnp.uint32, (EBLK, A), 1)
    n0 = (jnp.uint32(i * EBLK) + rows) * jnp.uint32(A) + cols
    bits = _threefry_bits(n0)
    fl = lax.bitcast_convert_type((bits >> 9) | jnp.uint32(0x3F800000),
                                  jnp.float32) - 1.0
    tiny = np.float32(np.finfo(np.float32).tiny)
    u = jnp.maximum(tiny, fl * (np.float32(1.0) - tiny) + tiny)
    g = -jnp.log(-jnp.log(u))

    mx = jnp.max(masked, axis=1, keepdims=True)
    ex = jnp.exp(masked - mx)
    S = jnp.sum(ex, axis=1, keepdims=True)
    p = ex / S
    ent = -jnp.sum(p * jnp.log(p + 1e-20), axis=1, keepdims=True)

    y = masked + g
    ymx = jnp.max(y, axis=1, keepdims=True)
    aidx = jax.lax.broadcasted_iota(jnp.int32, (EBLK, A), 1)
    idx = jnp.min(jnp.where(y == ymx, aidx, A), axis=1, keepdims=True)

    onehot = (aidx == idx)
    nr_ref[...] = jnp.sum(jnp.where(onehot, rs_ref[...], 0), axis=1,
                          keepdims=True)
    ne_ref[...] = jnp.sum(jnp.where(onehot, es_ref[...], 0), axis=1,
                          keepdims=True)
    ap_ref[...] = jnp.sum(jnp.where(onehot, p, 0.0), axis=1, keepdims=True)
    ent_ref[...] = ent


_epilogue = pl.pallas_call(
    _epilogue_body,
    grid=(B // EBLK,),
    in_specs=[
        pl.BlockSpec((EBLK, A), lambda i: (i, 0)),
        pl.BlockSpec((EBLK, A), lambda i: (i, 0)),
        pl.BlockSpec((EBLK, A), lambda i: (i, 0)),
        pl.BlockSpec((EBLK, A), lambda i: (i, 0)),
    ],
    out_specs=[
        pl.BlockSpec((EBLK, 1), lambda i: (i, 0)),
        pl.BlockSpec((EBLK, 1), lambda i: (i, 0)),
        pl.BlockSpec((EBLK, 1), lambda i: (i, 0)),
        pl.BlockSpec((EBLK, 1), lambda i: (i, 0)),
    ],
    out_shape=[
        jax.ShapeDtypeStruct((B, 1), jnp.float32),
        jax.ShapeDtypeStruct((B, 1), jnp.int32),
        jax.ShapeDtypeStruct((B, 1), jnp.int32),
        jax.ShapeDtypeStruct((B, 1), jnp.float32),
    ],
)


def kernel(e_t, H, r_q, r_space, e_space, action_mask, W1_w, W1_b, W2_w,
           W2_b, rel_table, ent_table):
    x2e, P = _prologue(e_t, H, r_q, W1_w, W1_b.reshape(1, -1), W2_w,
                       W2_b.reshape(1, -1), rel_table)
    scores = _sc_scores(x2e, P, e_space, r_space, ent_table)
    ap, nr, ne, ent = _epilogue(scores, r_space, e_space, action_mask)
    return ap[:, 0], nr[:, 0], ne[:, 0], ent[:, 0]
